# baseline (speedup 1.0000x reference)
import functools

import jax
import jax.numpy as jnp
from jax import lax
from jax.experimental import pallas as pl
from jax.experimental.pallas import tpu as pltpu
from jax.experimental.pallas import tpu_sc as plsc

N = 50000
E = 200000
D = 128
NP = 50176
NC, NS = 2, 16
NW = NC * NS
RPT = NP // NS
Q = 32
NQ = D // Q
GRID = 391

BA = 2000
NBA = E // BA
BC = 256
EP = 204800
EPW = EP // NW
NB = EPW // BC
ACCR = 50048
RPTA = ACCR // NS
PADDST = 50016

_mesh = plsc.VectorSubcoreMesh(
    core_axis_name="c", subcore_axis_name="s", num_cores=NC, num_subcores=NS)
_sc_params = pltpu.CompilerParams(use_tc_tiling_on_sc=False)


@functools.partial(
    pl.kernel,
    out_type=jax.ShapeDtypeStruct((NC, 6, NP), jnp.float32),
    mesh=_mesh,
    scratch_types=[
        pltpu.VMEM((BA,), jnp.int32),
        pltpu.VMEM((BA,), jnp.float32),
        pltpu.VMEM((1568,), jnp.float32),
    ] + [pltpu.VMEM_SHARED((NP,), jnp.float32) for _ in range(6)],
    compiler_params=_sc_params,
)
def _deg_call(e0, e1, e2, deg_hbm, idx_v, ones_v, zb, *daccs):
    c = lax.axis_index("c")
    s = lax.axis_index("s")
    w = s * NC + c

    def _fill_ones(i, carry):
        ones_v[pl.ds(i * 16, 16)] = jnp.full((16,), 1.0, jnp.float32)
        return carry

    lax.fori_loop(0, BA // 16, _fill_ones, 0)

    def _fill_z(i, carry):
        zb[pl.ds(i * 16, 16)] = jnp.zeros((16,), jnp.float32)
        return carry

    lax.fori_loop(0, 1568 // 16, _fill_z, 0)

    for dacc in daccs:
        pltpu.sync_copy(zb, dacc.at[pl.ds(s * RPT, 1568)])
        pltpu.sync_copy(zb, dacc.at[pl.ds(s * RPT + 1568, 1568)])
    plsc.subcore_barrier()

    for ridx, e_ref in enumerate((e0, e1, e2)):
        for dir_ in range(2):
            dacc = daccs[2 * ridx + dir_]

            def _body(k, carry, e_ref=e_ref, dacc=dacc, dir_=dir_):
                b = w + NW * k

                @pl.when(b < NBA)
                def _():
                    pltpu.sync_copy(e_ref.at[pl.ds(dir_ * EP + b * BA, BA)],
                                    idx_v)
                    pltpu.sync_copy(ones_v, dacc.at[idx_v], add=True)

                return carry

            lax.fori_loop(0, (NBA + NW - 1) // NW, _body, 0)
    plsc.subcore_barrier()

    for j, dacc in enumerate(daccs):
        pltpu.sync_copy(dacc.at[pl.ds(s * RPT, RPT)],
                        deg_hbm.at[c, j, pl.ds(s * RPT, RPT)])


def _mm_body(x_ref, w0_ref, w1_ref, w2_ref, deg_ref, z0_ref, z1_ref, z2_ref):
    xb = x_ref[...]
    deg = deg_ref[...]
    for r, (w_ref, z_ref) in enumerate(
            ((w0_ref, z0_ref), (w1_ref, z1_ref), (w2_ref, z2_ref))):
        d_out = deg[0, 2 * r] + deg[1, 2 * r]
        norm = lax.rsqrt(jnp.maximum(d_out, 1.0))
        z_ref[...] = jnp.dot(xb * norm[:, None], w_ref[...],
                             preferred_element_type=jnp.float32)


_mm_call = pl.pallas_call(
    _mm_body,
    grid=(GRID,),
    in_specs=[
        pl.BlockSpec((128, 128), lambda i: (i, 0)),
        pl.BlockSpec((128, 128), lambda i: (0, 0)),
        pl.BlockSpec((128, 128), lambda i: (0, 0)),
        pl.BlockSpec((128, 128), lambda i: (0, 0)),
        pl.BlockSpec((2, 6, 128), lambda i: (0, 0, i)),
    ],
    out_specs=[pl.BlockSpec((128, 128), lambda i: (i, 0)) for _ in range(3)],
    out_shape=[jax.ShapeDtypeStruct((N, D), jnp.float32) for _ in range(3)],
)


@functools.partial(
    pl.kernel,
    out_type=jax.ShapeDtypeStruct((NC, 3, NP, D), jnp.float32),
    mesh=_mesh,
    scratch_types=[
        pltpu.VMEM((EPW,), jnp.int32),
        pltpu.VMEM((EPW,), jnp.int32),
        pltpu.VMEM((BC, Q), jnp.float32),
        pltpu.VMEM((BC, Q), jnp.float32),
        pltpu.VMEM((BC,), jnp.int32),
        pltpu.VMEM((BC,), jnp.int32),
        pltpu.SemaphoreType.DMA,
        pltpu.SemaphoreType.DMA,
        pltpu.SemaphoreType.DMA,
        pltpu.SemaphoreType.DMA,
        pltpu.VMEM_SHARED((ACCR, Q), jnp.float32),
    ],
    compiler_params=_sc_params,
)
def _scat_call(e0, e1, e2, z0, z1, z2, p_hbm, src_all, dst_all,
               rows0, rows1, sd0, sd1, gs0, gs1, ss0, ss1, acc):
    c = lax.axis_index("c")
    s = lax.axis_index("s")
    w = s * NC + c
    rows = (rows0, rows1)
    sdst = (sd0, sd1)
    gsem = (gs0, gs1)
    ssem = (ss0, ss1)

    edges = (e0, e1, e2)
    ztabs = (z0, z1, z2)
    for r in range(3):
        e_ref = edges[r]
        ztab = ztabs[r]
        for q in range(NQ):
            def _fill_z(i, carry):
                for h in range(Q // 16):
                    rows0[i, pl.ds(h * 16, 16)] = jnp.zeros((16,),
                                                            jnp.float32)
                return carry

            lax.fori_loop(0, BC, _fill_z, 0)
            for jj in range(RPTA // BC):
                pltpu.sync_copy(rows0,
                                acc.at[pl.ds(s * RPTA + jj * BC, BC)])
            pltpu.sync_copy(
                rows0.at[pl.ds(0, RPTA % BC)],
                acc.at[pl.ds(s * RPTA + (RPTA // BC) * BC, RPTA % BC)])
            plsc.subcore_barrier()

            pltpu.sync_copy(e_ref.at[pl.ds(w * EPW, EPW)], src_all)
            pltpu.sync_copy(e_ref.at[pl.ds(EP + w * EPW, EPW)], dst_all)

            def _xf(t, carry):
                v = src_all[pl.ds(t * 16, 16)]
                src_all[pl.ds(t * 16, 16)] = v * 4 + q
                return carry

            lax.fori_loop(0, EPW // 16, _xf, 0)

            def _gather(k, p, ztab=ztab):
                pltpu.async_copy(
                    ztab.at[src_all.at[pl.ds(k * BC, BC)]], rows[p],
                    gsem[p])

            def _gwait(p, ztab=ztab):
                pltpu.make_async_copy(
                    ztab.at[src_all.at[pl.ds(0, BC)]], rows[p],
                    gsem[p]).wait()

            def _scatter(k, p):
                def _cp(t, carry):
                    sdst[p][pl.ds(t * 16, 16)] = dst_all[
                        pl.ds(k * BC + t * 16, 16)]
                    return carry

                lax.fori_loop(0, BC // 16, _cp, 0)
                pltpu.async_copy(rows[p], acc.at[sdst[p]], ssem[p],
                                 add=True)

            def _swait(p):
                pltpu.make_async_copy(rows[p], acc.at[sdst[p]],
                                      ssem[p]).wait()

            _gather(0, 0)
            _gather(1, 1)
            _gwait(0)
            _scatter(0, 0)

            def _pair(j, carry):
                _swait(0)
                _gather(2 * j + 2, 0)
                _gwait(1)
                _scatter(2 * j + 1, 1)
                _swait(1)
                _gather(2 * j + 3, 1)
                _gwait(0)
                _scatter(2 * j + 2, 0)
                return carry

            lax.fori_loop(0, 11, _pair, 0)
            _swait(0)
            _gather(24, 0)
            _gwait(1)
            _scatter(23, 1)
            _gwait(0)
            _scatter(24, 0)
            _swait(1)
            _swait(0)

            plsc.subcore_barrier()

            pltpu.sync_copy(
                acc.at[pl.ds(s * RPTA, RPTA)],
                p_hbm.at[c, r, pl.ds(s * RPTA, RPTA), pl.ds(q * Q, Q)])
            plsc.subcore_barrier()


def _comb_body(p_ref, deg_ref, o_ref):
    deg = deg_ref[...]
    p = p_ref[...]
    acc = jnp.zeros((128, D), jnp.float32)
    for r in range(3):
        d_in = deg[0, 2 * r + 1] + deg[1, 2 * r + 1]
        norm = lax.rsqrt(jnp.maximum(d_in, 1.0))
        acc = acc + (p[0, r] + p[1, r]) * norm[:, None]
    o_ref[...] = acc


_comb_call = pl.pallas_call(
    _comb_body,
    grid=(GRID,),
    in_specs=[
        pl.BlockSpec((2, 3, 128, 128), lambda i: (0, 0, i, 0)),
        pl.BlockSpec((2, 6, 128), lambda i: (0, 0, i)),
    ],
    out_specs=pl.BlockSpec((128, 128), lambda i: (i, 0)),
    out_shape=jax.ShapeDtypeStruct((N, D), jnp.float32),
)


def _pad_edges(e):
    pad_src = jnp.zeros((EP - E,), jnp.int32)
    pad_dst = jnp.full((EP - E,), PADDST, jnp.int32)
    return jnp.concatenate([e[0], pad_src, e[1], pad_dst])


def kernel(x, edge_index_r0, edge_index_r1, edge_index_r2, W_r0, W_r1, W_r2):
    e0 = _pad_edges(edge_index_r0)
    e1 = _pad_edges(edge_index_r1)
    e2 = _pad_edges(edge_index_r2)
    deg = _deg_call(e0, e1, e2)
    z0, z1, z2 = _mm_call(x, W_r0, W_r1, W_r2, deg)
    zq = [z.reshape(4 * N, Q) for z in (z0, z1, z2)]
    p = _scat_call(e0, e1, e2, *zq)
    return _comb_call(p, deg)


# speedup vs baseline: 1.8911x; 1.8911x over previous
import functools

import jax
import jax.numpy as jnp
from jax import lax
from jax.experimental import pallas as pl
from jax.experimental.pallas import tpu as pltpu
from jax.experimental.pallas import tpu_sc as plsc

N = 50000
E = 200000
D = 128
NP = 50176
NC, NS = 2, 16
NW = NC * NS
RPT = NP // NS
Q = 32
NQ = D // Q
BR = 512
GRIDB = NP // BR

BA = 2000
NBA = E // BA
BC = 320
NBC = E // BC

_mesh = plsc.VectorSubcoreMesh(
    core_axis_name="c", subcore_axis_name="s", num_cores=NC, num_subcores=NS)
_sc_params = pltpu.CompilerParams(use_tc_tiling_on_sc=False)


@functools.partial(
    pl.kernel,
    out_type=jax.ShapeDtypeStruct((NC, 6, NP), jnp.float32),
    mesh=_mesh,
    scratch_types=[
        pltpu.VMEM((BA,), jnp.int32),
        pltpu.VMEM((BA,), jnp.float32),
        pltpu.VMEM((1568,), jnp.float32),
    ] + [pltpu.VMEM_SHARED((NP,), jnp.float32) for _ in range(6)],
    compiler_params=_sc_params,
)
def _deg_call(e0, e1, e2, deg_hbm, idx_v, ones_v, zb, *daccs):
    c = lax.axis_index("c")
    s = lax.axis_index("s")
    w = s * NC + c

    def _fill_ones(i, carry):
        ones_v[pl.ds(i * 16, 16)] = jnp.full((16,), 1.0, jnp.float32)
        return carry

    lax.fori_loop(0, BA // 16, _fill_ones, 0)

    def _fill_z(i, carry):
        zb[pl.ds(i * 16, 16)] = jnp.zeros((16,), jnp.float32)
        return carry

    lax.fori_loop(0, 1568 // 16, _fill_z, 0)

    for dacc in daccs:
        pltpu.sync_copy(zb, dacc.at[pl.ds(s * RPT, 1568)])
        pltpu.sync_copy(zb, dacc.at[pl.ds(s * RPT + 1568, 1568)])
    plsc.subcore_barrier()

    for ridx, e_ref in enumerate((e0, e1, e2)):
        for dir_ in range(2):
            dacc = daccs[2 * ridx + dir_]

            def _body(k, carry, e_ref=e_ref, dacc=dacc, dir_=dir_):
                b = w + NW * k

                @pl.when(b < NBA)
                def _():
                    pltpu.sync_copy(e_ref.at[pl.ds(dir_ * E + b * BA, BA)],
                                    idx_v)
                    pltpu.sync_copy(ones_v, dacc.at[idx_v], add=True)

                return carry

            lax.fori_loop(0, (NBA + NW - 1) // NW, _body, 0)
    plsc.subcore_barrier()

    for j, dacc in enumerate(daccs):
        pltpu.sync_copy(dacc.at[pl.ds(s * RPT, RPT)],
                        deg_hbm.at[c, j, pl.ds(s * RPT, RPT)])


def _mm_body(x_ref, w0_ref, w1_ref, w2_ref, deg_ref, z0_ref, z1_ref, z2_ref):
    xb = x_ref[...]
    deg = deg_ref[...]
    for r, (w_ref, z_ref) in enumerate(
            ((w0_ref, z0_ref), (w1_ref, z1_ref), (w2_ref, z2_ref))):
        d_out = deg[0, 2 * r] + deg[1, 2 * r]
        norm = lax.rsqrt(jnp.maximum(d_out, 1.0))
        z_ref[...] = jnp.dot(xb * norm[:, None], w_ref[...],
                             preferred_element_type=jnp.float32)


_mm_call = pl.pallas_call(
    _mm_body,
    grid=(GRIDB,),
    in_specs=[
        pl.BlockSpec((BR, 128), lambda i: (i, 0)),
        pl.BlockSpec((128, 128), lambda i: (0, 0)),
        pl.BlockSpec((128, 128), lambda i: (0, 0)),
        pl.BlockSpec((128, 128), lambda i: (0, 0)),
        pl.BlockSpec((2, 6, BR), lambda i: (0, 0, i)),
    ],
    out_specs=[pl.BlockSpec((BR, 128), lambda i: (i, 0)) for _ in range(3)],
    out_shape=[jax.ShapeDtypeStruct((N, D), jnp.float32) for _ in range(3)],
)


@functools.partial(
    pl.kernel,
    out_type=jax.ShapeDtypeStruct((NC, 3, NP, D), jnp.float32),
    mesh=_mesh,
    scratch_types=[
        pltpu.VMEM((BC,), jnp.int32),
        pltpu.VMEM((BC,), jnp.int32),
        pltpu.VMEM((BC,), jnp.int32),
        pltpu.VMEM((BC,), jnp.int32),
        pltpu.VMEM((BC, Q), jnp.float32),
        pltpu.VMEM((BC, Q), jnp.float32),
        pltpu.VMEM((196, Q), jnp.float32),
        pltpu.SemaphoreType.DMA,
        pltpu.SemaphoreType.DMA,
        pltpu.VMEM_SHARED((NP, Q), jnp.float32),
    ],
    compiler_params=_sc_params,
)
def _scat_call(e0, e1, e2, z0, z1, z2, p_hbm, src0, src1, dst0, dst1,
               rows0, rows1, zb, sem0, sem1, acc):
    c = lax.axis_index("c")
    s = lax.axis_index("s")
    w = s * NC + c
    srcs = (src0, src1)
    dsts = (dst0, dst1)
    rows = (rows0, rows1)
    sems = (sem0, sem1)

    def _fill_z(i, carry):
        zb[i, pl.ds(0, 16)] = jnp.zeros((16,), jnp.float32)
        zb[i, pl.ds(16, 16)] = jnp.zeros((16,), jnp.float32)
        return carry

    lax.fori_loop(0, 196, _fill_z, 0)

    edges = (e0, e1, e2)
    ztabs = (z0, z1, z2)
    for r in range(3):
        for q in range(NQ):
            for jj in range(16):
                pltpu.sync_copy(zb, acc.at[pl.ds(s * RPT + jj * 196, 196)])
            plsc.subcore_barrier()

            ztab = ztabs[r]
            e_ref = edges[r]

            def _fetch(b, p, ztab=ztab, e_ref=e_ref):
                pltpu.sync_copy(e_ref.at[pl.ds(b * BC, BC)], srcs[p])
                pltpu.sync_copy(e_ref.at[pl.ds(E + b * BC, BC)], dsts[p])

                def _xf(t, carry):
                    v = srcs[p][pl.ds(t * 16, 16)]
                    srcs[p][pl.ds(t * 16, 16)] = v * 4 + q
                    return carry

                lax.fori_loop(0, BC // 16, _xf, 0)
                pltpu.async_copy(ztab.at[srcs[p]], rows[p], sems[p])

            def _drain(p, ztab=ztab):
                pltpu.make_async_copy(ztab.at[srcs[p]], rows[p],
                                      sems[p]).wait()
                pltpu.sync_copy(rows[p], acc.at[dsts[p]], add=True)

            _fetch(w, 0)

            def _pair(j, carry):
                _fetch(w + NW * (2 * j + 1), 1)
                _drain(0)
                _fetch(w + NW * (2 * j + 2), 0)
                _drain(1)
                return carry

            lax.fori_loop(0, 9, _pair, 0)
            _drain(0)

            @pl.when(w < 17)
            def _():
                _fetch(608 + w, 1)
                _drain(1)

            plsc.subcore_barrier()

            pltpu.sync_copy(
                acc.at[pl.ds(s * RPT, RPT)],
                p_hbm.at[c, r, pl.ds(s * RPT, RPT), pl.ds(q * Q, Q)])
            plsc.subcore_barrier()


def _comb_body(p_ref, deg_ref, o_ref):
    deg = deg_ref[...]
    p = p_ref[...]
    acc = jnp.zeros((BR, D), jnp.float32)
    for r in range(3):
        d_in = deg[0, 2 * r + 1] + deg[1, 2 * r + 1]
        norm = lax.rsqrt(jnp.maximum(d_in, 1.0))
        acc = acc + (p[0, r] + p[1, r]) * norm[:, None]
    o_ref[...] = acc


_comb_call = pl.pallas_call(
    _comb_body,
    grid=(GRIDB,),
    in_specs=[
        pl.BlockSpec((2, 3, BR, 128), lambda i: (0, 0, i, 0)),
        pl.BlockSpec((2, 6, BR), lambda i: (0, 0, i)),
    ],
    out_specs=pl.BlockSpec((BR, 128), lambda i: (i, 0)),
    out_shape=jax.ShapeDtypeStruct((N, D), jnp.float32),
)


def kernel(x, edge_index_r0, edge_index_r1, edge_index_r2, W_r0, W_r1, W_r2):
    e0 = edge_index_r0.reshape(-1)
    e1 = edge_index_r1.reshape(-1)
    e2 = edge_index_r2.reshape(-1)
    deg = _deg_call(e0, e1, e2)
    z0, z1, z2 = _mm_call(x, W_r0, W_r1, W_r2, deg)
    zq = [z.reshape(4 * N, Q) for z in (z0, z1, z2)]
    p = _scat_call(e0, e1, e2, *zq)
    return _comb_call(p, deg)


# speedup vs baseline: 1.9498x; 1.0310x over previous
import functools

import jax
import jax.numpy as jnp
from jax import lax
from jax.experimental import pallas as pl
from jax.experimental.pallas import tpu as pltpu
from jax.experimental.pallas import tpu_sc as plsc

N = 50000
E = 200000
D = 128
NP = 50176
NC, NS = 2, 16
NW = NC * NS
RPT = NP // NS
Q = 32
NQ = D // Q
BR = 512
GRIDB = NP // BR

BA = 2000
NBA = E // BA
BC = 400
NBC = E // BC

_mesh = plsc.VectorSubcoreMesh(
    core_axis_name="c", subcore_axis_name="s", num_cores=NC, num_subcores=NS)
_sc_params = pltpu.CompilerParams(use_tc_tiling_on_sc=False)


@functools.partial(
    pl.kernel,
    out_type=jax.ShapeDtypeStruct((NC, 6, NP), jnp.float32),
    mesh=_mesh,
    scratch_types=[
        pltpu.VMEM((BA,), jnp.int32),
        pltpu.VMEM((BA,), jnp.float32),
        pltpu.VMEM((1568,), jnp.float32),
    ] + [pltpu.VMEM_SHARED((NP,), jnp.float32) for _ in range(6)],
    compiler_params=_sc_params,
)
def _deg_call(e0, e1, e2, deg_hbm, idx_v, ones_v, zb, *daccs):
    c = lax.axis_index("c")
    s = lax.axis_index("s")
    w = s * NC + c

    def _fill_ones(i, carry):
        ones_v[pl.ds(i * 16, 16)] = jnp.full((16,), 1.0, jnp.float32)
        return carry

    lax.fori_loop(0, BA // 16, _fill_ones, 0)

    def _fill_z(i, carry):
        zb[pl.ds(i * 16, 16)] = jnp.zeros((16,), jnp.float32)
        return carry

    lax.fori_loop(0, 1568 // 16, _fill_z, 0)

    for dacc in daccs:
        pltpu.sync_copy(zb, dacc.at[pl.ds(s * RPT, 1568)])
        pltpu.sync_copy(zb, dacc.at[pl.ds(s * RPT + 1568, 1568)])
    plsc.subcore_barrier()

    for ridx, e_ref in enumerate((e0, e1, e2)):
        for dir_ in range(2):
            dacc = daccs[2 * ridx + dir_]

            def _body(k, carry, e_ref=e_ref, dacc=dacc, dir_=dir_):
                b = w + NW * k

                @pl.when(b < NBA)
                def _():
                    pltpu.sync_copy(e_ref.at[pl.ds(dir_ * E + b * BA, BA)],
                                    idx_v)
                    pltpu.sync_copy(ones_v, dacc.at[idx_v], add=True)

                return carry

            lax.fori_loop(0, (NBA + NW - 1) // NW, _body, 0)
    plsc.subcore_barrier()

    for j, dacc in enumerate(daccs):
        pltpu.sync_copy(dacc.at[pl.ds(s * RPT, RPT)],
                        deg_hbm.at[c, j, pl.ds(s * RPT, RPT)])


def _mm_body(x_ref, w0_ref, w1_ref, w2_ref, deg_ref, z0_ref, z1_ref, z2_ref):
    xb = x_ref[...]
    deg = deg_ref[...]
    for r, (w_ref, z_ref) in enumerate(
            ((w0_ref, z0_ref), (w1_ref, z1_ref), (w2_ref, z2_ref))):
        d_out = deg[0, 2 * r] + deg[1, 2 * r]
        norm = lax.rsqrt(jnp.maximum(d_out, 1.0))
        z_ref[...] = jnp.dot(xb * norm[:, None], w_ref[...],
                             preferred_element_type=jnp.float32)


_mm_call = pl.pallas_call(
    _mm_body,
    grid=(GRIDB,),
    in_specs=[
        pl.BlockSpec((BR, 128), lambda i: (i, 0)),
        pl.BlockSpec((128, 128), lambda i: (0, 0)),
        pl.BlockSpec((128, 128), lambda i: (0, 0)),
        pl.BlockSpec((128, 128), lambda i: (0, 0)),
        pl.BlockSpec((2, 6, BR), lambda i: (0, 0, i)),
    ],
    out_specs=[pl.BlockSpec((BR, 128), lambda i: (i, 0)) for _ in range(3)],
    out_shape=[jax.ShapeDtypeStruct((N, D), jnp.float32) for _ in range(3)],
)


@functools.partial(
    pl.kernel,
    out_type=jax.ShapeDtypeStruct((NC, 3, NP, D), jnp.float32),
    mesh=_mesh,
    scratch_types=[
        pltpu.VMEM((BC,), jnp.int32),
        pltpu.VMEM((BC,), jnp.int32),
        pltpu.VMEM((BC,), jnp.int32),
        pltpu.VMEM((BC,), jnp.int32),
        pltpu.VMEM((BC, Q), jnp.float32),
        pltpu.VMEM((BC, Q), jnp.float32),
        pltpu.VMEM((98, Q), jnp.float32),
        pltpu.SemaphoreType.DMA,
        pltpu.SemaphoreType.DMA,
        pltpu.VMEM_SHARED((NP, Q), jnp.float32),
    ],
    compiler_params=_sc_params,
)
def _scat_call(e0, e1, e2, z0, z1, z2, p_hbm, src0, src1, dst0, dst1,
               rows0, rows1, zb, sem0, sem1, acc):
    c = lax.axis_index("c")
    s = lax.axis_index("s")
    w = s * NC + c
    srcs = (src0, src1)
    dsts = (dst0, dst1)
    rows = (rows0, rows1)
    sems = (sem0, sem1)

    def _fill_z(i, carry):
        zb[i, pl.ds(0, 16)] = jnp.zeros((16,), jnp.float32)
        zb[i, pl.ds(16, 16)] = jnp.zeros((16,), jnp.float32)
        return carry

    lax.fori_loop(0, 98, _fill_z, 0)

    edges = (e0, e1, e2)
    ztabs = (z0, z1, z2)
    for r in range(3):
        for q in range(NQ):
            for jj in range(32):
                pltpu.sync_copy(zb, acc.at[pl.ds(s * RPT + jj * 98, 98)])
            plsc.subcore_barrier()

            ztab = ztabs[r]
            e_ref = edges[r]

            def _fetch(b, p, ztab=ztab, e_ref=e_ref):
                pltpu.sync_copy(e_ref.at[pl.ds(b * BC, BC)], srcs[p])
                pltpu.sync_copy(e_ref.at[pl.ds(E + b * BC, BC)], dsts[p])

                def _xf(t, carry):
                    v = srcs[p][pl.ds(t * 16, 16)]
                    srcs[p][pl.ds(t * 16, 16)] = v * 4 + q
                    return carry

                lax.fori_loop(0, BC // 16, _xf, 0)
                pltpu.async_copy(ztab.at[srcs[p]], rows[p], sems[p])

            def _drain(p, ztab=ztab):
                pltpu.make_async_copy(ztab.at[srcs[p]], rows[p],
                                      sems[p]).wait()
                pltpu.sync_copy(rows[p], acc.at[dsts[p]], add=True)

            _fetch(w, 0)

            def _pair(j, carry):
                _fetch(w + NW * (2 * j + 1), 1)
                _drain(0)
                _fetch(w + NW * (2 * j + 2), 0)
                _drain(1)
                return carry

            lax.fori_loop(0, 7, _pair, 0)
            _drain(0)

            @pl.when(w < 20)
            def _():
                _fetch(480 + w, 1)
                _drain(1)

            plsc.subcore_barrier()

            pltpu.sync_copy(
                acc.at[pl.ds(s * RPT, RPT)],
                p_hbm.at[c, r, pl.ds(s * RPT, RPT), pl.ds(q * Q, Q)])
            plsc.subcore_barrier()


def _comb_body(p_ref, deg_ref, o_ref):
    deg = deg_ref[...]
    p = p_ref[...]
    acc = jnp.zeros((BR, D), jnp.float32)
    for r in range(3):
        d_in = deg[0, 2 * r + 1] + deg[1, 2 * r + 1]
        norm = lax.rsqrt(jnp.maximum(d_in, 1.0))
        acc = acc + (p[0, r] + p[1, r]) * norm[:, None]
    o_ref[...] = acc


_comb_call = pl.pallas_call(
    _comb_body,
    grid=(GRIDB,),
    in_specs=[
        pl.BlockSpec((2, 3, BR, 128), lambda i: (0, 0, i, 0)),
        pl.BlockSpec((2, 6, BR), lambda i: (0, 0, i)),
    ],
    out_specs=pl.BlockSpec((BR, 128), lambda i: (i, 0)),
    out_shape=jax.ShapeDtypeStruct((N, D), jnp.float32),
)


def kernel(x, edge_index_r0, edge_index_r1, edge_index_r2, W_r0, W_r1, W_r2):
    e0 = edge_index_r0.reshape(-1)
    e1 = edge_index_r1.reshape(-1)
    e2 = edge_index_r2.reshape(-1)
    deg = _deg_call(e0, e1, e2)
    z0, z1, z2 = _mm_call(x, W_r0, W_r1, W_r2, deg)
    zq = [z.reshape(4 * N, Q) for z in (z0, z1, z2)]
    p = _scat_call(e0, e1, e2, *zq)
    return _comb_call(p, deg)


# speedup vs baseline: 2.0621x; 1.0576x over previous
import functools

import jax
import jax.numpy as jnp
from jax import lax
from jax.experimental import pallas as pl
from jax.experimental.pallas import tpu as pltpu
from jax.experimental.pallas import tpu_sc as plsc

N = 50000
E = 200000
D = 128
NP = 50176
NC, NS = 2, 16
NW = NC * NS
RPT = NP // NS
Q = 32
NQ = D // Q
BR = 1024
GRIDB = NP // BR

BA = 2000
NBA = E // BA
BC = 400
NBC = E // BC

_mesh = plsc.VectorSubcoreMesh(
    core_axis_name="c", subcore_axis_name="s", num_cores=NC, num_subcores=NS)
_sc_params = pltpu.CompilerParams(use_tc_tiling_on_sc=False)


@functools.partial(
    pl.kernel,
    out_type=jax.ShapeDtypeStruct((NC, 6, NP), jnp.float32),
    mesh=_mesh,
    scratch_types=[
        pltpu.VMEM((BA,), jnp.int32),
        pltpu.VMEM((BA,), jnp.float32),
        pltpu.VMEM((1568,), jnp.float32),
    ] + [pltpu.VMEM_SHARED((NP,), jnp.float32) for _ in range(6)],
    compiler_params=_sc_params,
)
def _deg_call(e0, e1, e2, deg_hbm, idx_v, ones_v, zb, *daccs):
    c = lax.axis_index("c")
    s = lax.axis_index("s")
    w = s * NC + c

    def _fill_ones(i, carry):
        ones_v[pl.ds(i * 16, 16)] = jnp.full((16,), 1.0, jnp.float32)
        return carry

    lax.fori_loop(0, BA // 16, _fill_ones, 0)

    def _fill_z(i, carry):
        zb[pl.ds(i * 16, 16)] = jnp.zeros((16,), jnp.float32)
        return carry

    lax.fori_loop(0, 1568 // 16, _fill_z, 0)

    for dacc in daccs:
        pltpu.sync_copy(zb, dacc.at[pl.ds(s * RPT, 1568)])
        pltpu.sync_copy(zb, dacc.at[pl.ds(s * RPT + 1568, 1568)])
    plsc.subcore_barrier()

    for ridx, e_ref in enumerate((e0, e1, e2)):
        for dir_ in range(2):
            dacc = daccs[2 * ridx + dir_]

            def _body(k, carry, e_ref=e_ref, dacc=dacc, dir_=dir_):
                b = w + NW * k

                @pl.when(b < NBA)
                def _():
                    pltpu.sync_copy(e_ref.at[pl.ds(dir_ * E + b * BA, BA)],
                                    idx_v)
                    pltpu.sync_copy(ones_v, dacc.at[idx_v], add=True)

                return carry

            lax.fori_loop(0, (NBA + NW - 1) // NW, _body, 0)
    plsc.subcore_barrier()

    for j, dacc in enumerate(daccs):
        pltpu.sync_copy(dacc.at[pl.ds(s * RPT, RPT)],
                        deg_hbm.at[c, j, pl.ds(s * RPT, RPT)])


def _mm_body(x_ref, w0_ref, w1_ref, w2_ref, deg_ref, z0_ref, z1_ref, z2_ref):
    xb = x_ref[...]
    deg = deg_ref[...]
    for r, (w_ref, z_ref) in enumerate(
            ((w0_ref, z0_ref), (w1_ref, z1_ref), (w2_ref, z2_ref))):
        d_out = deg[0, 2 * r] + deg[1, 2 * r]
        norm = lax.rsqrt(jnp.maximum(d_out, 1.0))
        z_ref[...] = jnp.dot(xb * norm[:, None], w_ref[...],
                             preferred_element_type=jnp.float32)


_mm_call = pl.pallas_call(
    _mm_body,
    grid=(GRIDB,),
    in_specs=[
        pl.BlockSpec((BR, 128), lambda i: (i, 0)),
        pl.BlockSpec((128, 128), lambda i: (0, 0)),
        pl.BlockSpec((128, 128), lambda i: (0, 0)),
        pl.BlockSpec((128, 128), lambda i: (0, 0)),
        pl.BlockSpec((2, 6, BR), lambda i: (0, 0, i)),
    ],
    out_specs=[pl.BlockSpec((BR, 128), lambda i: (i, 0)) for _ in range(3)],
    out_shape=[jax.ShapeDtypeStruct((N, D), jnp.float32) for _ in range(3)],
)


@functools.partial(
    pl.kernel,
    out_type=jax.ShapeDtypeStruct((NC, 3, NP, D), jnp.float32),
    mesh=_mesh,
    scratch_types=[
        pltpu.VMEM((BC,), jnp.int32),
        pltpu.VMEM((BC,), jnp.int32),
        pltpu.VMEM((BC,), jnp.int32),
        pltpu.VMEM((BC,), jnp.int32),
        pltpu.VMEM((BC, Q), jnp.float32),
        pltpu.VMEM((BC, Q), jnp.float32),
        pltpu.VMEM((98, Q), jnp.float32),
        pltpu.SemaphoreType.DMA,
        pltpu.SemaphoreType.DMA,
        pltpu.VMEM_SHARED((NP, Q), jnp.float32),
    ],
    compiler_params=_sc_params,
)
def _scat_call(e0, e1, e2, z0, z1, z2, p_hbm, src0, src1, dst0, dst1,
               rows0, rows1, zb, sem0, sem1, acc):
    c = lax.axis_index("c")
    s = lax.axis_index("s")
    w = s * NC + c
    srcs = (src0, src1)
    dsts = (dst0, dst1)
    rows = (rows0, rows1)
    sems = (sem0, sem1)

    def _fill_z(i, carry):
        zb[i, pl.ds(0, 16)] = jnp.zeros((16,), jnp.float32)
        zb[i, pl.ds(16, 16)] = jnp.zeros((16,), jnp.float32)
        return carry

    lax.fori_loop(0, 98, _fill_z, 0)

    edges = (e0, e1, e2)
    ztabs = (z0, z1, z2)
    for r in range(3):
        for q in range(NQ):
            for jj in range(32):
                pltpu.sync_copy(zb, acc.at[pl.ds(s * RPT + jj * 98, 98)])
            plsc.subcore_barrier()

            ztab = ztabs[r]
            e_ref = edges[r]

            def _fetch(b, p, ztab=ztab, e_ref=e_ref):
                pltpu.sync_copy(e_ref.at[pl.ds(b * BC, BC)], srcs[p])
                pltpu.sync_copy(e_ref.at[pl.ds(E + b * BC, BC)], dsts[p])

                def _xf(t, carry):
                    v = srcs[p][pl.ds(t * 16, 16)]
                    srcs[p][pl.ds(t * 16, 16)] = v * 4 + q
                    return carry

                lax.fori_loop(0, BC // 16, _xf, 0)
                pltpu.async_copy(ztab.at[srcs[p]], rows[p], sems[p])

            def _drain(p, ztab=ztab):
                pltpu.make_async_copy(ztab.at[srcs[p]], rows[p],
                                      sems[p]).wait()
                pltpu.sync_copy(rows[p], acc.at[dsts[p]], add=True)

            _fetch(w, 0)

            def _pair(j, carry):
                _fetch(w + NW * (2 * j + 1), 1)
                _drain(0)
                _fetch(w + NW * (2 * j + 2), 0)
                _drain(1)
                return carry

            lax.fori_loop(0, 7, _pair, 0)
            _drain(0)

            @pl.when(w < 20)
            def _():
                _fetch(480 + w, 1)
                _drain(1)

            plsc.subcore_barrier()

            pltpu.sync_copy(
                acc.at[pl.ds(s * RPT, RPT)],
                p_hbm.at[c, r, pl.ds(s * RPT, RPT), pl.ds(q * Q, Q)])
            plsc.subcore_barrier()


def _comb_body(p_ref, deg_ref, o_ref):
    deg = deg_ref[...]
    p = p_ref[...]
    acc = jnp.zeros((BR, D), jnp.float32)
    for r in range(3):
        d_in = deg[0, 2 * r + 1] + deg[1, 2 * r + 1]
        norm = lax.rsqrt(jnp.maximum(d_in, 1.0))
        acc = acc + (p[0, r] + p[1, r]) * norm[:, None]
    o_ref[...] = acc


_comb_call = pl.pallas_call(
    _comb_body,
    grid=(GRIDB,),
    in_specs=[
        pl.BlockSpec((2, 3, BR, 128), lambda i: (0, 0, i, 0)),
        pl.BlockSpec((2, 6, BR), lambda i: (0, 0, i)),
    ],
    out_specs=pl.BlockSpec((BR, 128), lambda i: (i, 0)),
    out_shape=jax.ShapeDtypeStruct((N, D), jnp.float32),
)


def kernel(x, edge_index_r0, edge_index_r1, edge_index_r2, W_r0, W_r1, W_r2):
    e0 = edge_index_r0.reshape(-1)
    e1 = edge_index_r1.reshape(-1)
    e2 = edge_index_r2.reshape(-1)
    deg = _deg_call(e0, e1, e2)
    z0, z1, z2 = _mm_call(x, W_r0, W_r1, W_r2, deg)
    zq = [z.reshape(4 * N, Q) for z in (z0, z1, z2)]
    p = _scat_call(e0, e1, e2, *zq)
    return _comb_call(p, deg)
